# baseline (speedup 1.0000x reference)
import jax
import jax.numpy as jnp
from jax.experimental import pallas as pl
from jax.experimental.pallas import tpu as pltpu

_BM = 4096
_D = 256
_GRID = 4


def _body(idx_ref, x_hbm, pos_ref, out_hbm, buf0, buf1, pos_full,
          in_sems, out_sems):
    i = pl.program_id(0)

    def in_cp(c, buf, sem):
        return pltpu.make_async_copy(
            x_hbm.at[pl.ds(c * _BM, _BM), :, :],
            buf.at[:, :, pl.ds(0, _D)],
            sem,
        )

    def out_cp(c, buf, sem):
        return pltpu.make_async_copy(
            buf,
            out_hbm.at[pl.ds(c * _BM, _BM), :, :],
            sem,
        )

    def ring(c, buf, in_sem, out_sem):
        in_cp(c, buf, in_sem).wait()
        buf[:, 0, _D:2 * _D] = pos_full[...]
        out_cp(c, buf, out_sem).start()

    @pl.when(i == 0)
    def _():
        row = idx_ref[0] % 8
        pos_full[...] = jnp.broadcast_to(
            pos_ref[0, pl.ds(row, 1), :], (_BM, _D))
        in_cp(0, buf0, in_sems.at[0]).start()
        in_cp(1, buf1, in_sems.at[1]).start()

    @pl.when(i % 2 == 0)
    def _():
        ring(i, buf0, in_sems.at[0], out_sems.at[0])

    @pl.when(i % 2 == 1)
    def _():
        ring(i, buf1, in_sems.at[1], out_sems.at[1])

    @pl.when((i >= 1) & (i < _GRID - 1))
    def _():
        p = i - 1

        @pl.when(p % 2 == 0)
        def _():
            out_cp(p, buf0, out_sems.at[0]).wait()
            in_cp(p + 2, buf0, in_sems.at[0]).start()

        @pl.when(p % 2 == 1)
        def _():
            out_cp(p, buf1, out_sems.at[1]).wait()
            in_cp(p + 2, buf1, in_sems.at[1]).start()

    @pl.when(i == _GRID - 1)
    def _():
        pb = _GRID - 2
        lb = _GRID - 1
        out_cp(pb, buf0 if pb % 2 == 0 else buf1,
               out_sems.at[pb % 2]).wait()
        out_cp(lb, buf0 if lb % 2 == 0 else buf1,
               out_sems.at[lb % 2]).wait()


def kernel(x, pos_table, index):
    B, _, D = x.shape
    idx = jnp.asarray(index, jnp.int32).reshape(1)
    return pl.pallas_call(
        _body,
        grid_spec=pltpu.PrefetchScalarGridSpec(
            num_scalar_prefetch=1,
            grid=(_GRID,),
            in_specs=[
                pl.BlockSpec(memory_space=pltpu.HBM),
                pl.BlockSpec((1, 8, D), lambda i, s: (0, s[0] // 8, 0)),
            ],
            out_specs=pl.BlockSpec(memory_space=pltpu.HBM),
            scratch_shapes=[
                pltpu.VMEM((_BM, 1, 2 * _D), jnp.float32),
                pltpu.VMEM((_BM, 1, 2 * _D), jnp.float32),
                pltpu.VMEM((_BM, _D), jnp.float32),
                pltpu.SemaphoreType.DMA((2,)),
                pltpu.SemaphoreType.DMA((2,)),
            ],
        ),
        out_shape=jax.ShapeDtypeStruct((B, 1, 2 * D), jnp.float32),
        compiler_params=pltpu.CompilerParams(
            dimension_semantics=("arbitrary",),
        ),
    )(idx, x, pos_table)


# speedup vs baseline: 1.2592x; 1.2592x over previous
import jax
import jax.numpy as jnp
from jax.experimental import pallas as pl
from jax.experimental.pallas import tpu as pltpu

_BM = 8192
_D = 256


def _body(idx_ref, x_hbm, pos_ref, out_ref, pos_full, sem):
    i = pl.program_id(0)
    cp = pltpu.make_async_copy(
        x_hbm.at[pl.ds(i * _BM, _BM), :, :],
        out_ref.at[:, :, pl.ds(0, _D)],
        sem,
    )
    cp.start()

    @pl.when(i == 0)
    def _():
        row = idx_ref[0] % 8
        pos_full[...] = jnp.broadcast_to(
            pos_ref[0, pl.ds(row, 1), :], (_BM, _D))

    out_ref[:, 0, _D:2 * _D] = pos_full[...]
    cp.wait()


def kernel(x, pos_table, index):
    B, _, D = x.shape
    grid = B // _BM
    idx = jnp.asarray(index, jnp.int32).reshape(1)
    return pl.pallas_call(
        _body,
        grid_spec=pltpu.PrefetchScalarGridSpec(
            num_scalar_prefetch=1,
            grid=(grid,),
            in_specs=[
                pl.BlockSpec(memory_space=pltpu.HBM),
                pl.BlockSpec((1, 8, D), lambda i, s: (0, s[0] // 8, 0)),
            ],
            out_specs=pl.BlockSpec((_BM, 1, 2 * D), lambda i, s: (i, 0, 0)),
            scratch_shapes=[
                pltpu.VMEM((_BM, _D), jnp.float32),
                pltpu.SemaphoreType.DMA,
            ],
        ),
        out_shape=jax.ShapeDtypeStruct((B, 1, 2 * D), jnp.float32),
        compiler_params=pltpu.CompilerParams(
            dimension_semantics=("parallel",),
        ),
    )(idx, x, pos_table)
